# baseline (speedup 1.0000x reference)
import functools

import jax
import jax.numpy as jnp
from jax import lax
from jax.experimental import pallas as pl
from jax.experimental.pallas import tpu as pltpu
from jax.experimental.pallas import tpu_sc as plsc


def _sc_gather(tables, dim, Hs, Ws):
    lanes = 16
    nrows = tables.shape[0]
    mesh = plsc.VectorSubcoreMesh(
        core_axis_name="c", subcore_axis_name="s", num_cores=1)

    @functools.partial(
        pl.kernel,
        mesh=mesh,
        compiler_params=pltpu.CompilerParams(needs_layout_passes=False),
        out_type=(
            jax.ShapeDtypeStruct((Hs, dim, Hs), jnp.float32),
            jax.ShapeDtypeStruct((Ws, dim, Ws), jnp.float32),
        ),
        scratch_types=[
            pltpu.VMEM((nrows, dim), jnp.float32),
            pltpu.VMEM((dim, Hs), jnp.float32),
            pltpu.VMEM((dim, Ws), jnp.float32),
            pltpu.SemaphoreType.DMA,
            pltpu.SemaphoreType.DMA,
        ],
    )
    def gather_kernel(tbl_hbm, eh_hbm, ew_hbm, tbl_v, ehs, ews, sem_h, sem_w):
        sid = lax.axis_index("s")
        pltpu.async_copy(tbl_hbm, tbl_v, sem_h).wait()
        lane = lax.iota(jnp.int32, lanes)
        for half in range(2):
            wid = sid + 16 * half
            for c in range(Hs // lanes):
                r = (Hs - 1 + wid - lanes * c) - lane
                for d in range(dim):
                    dv = jnp.full((lanes,), d, jnp.int32)
                    ehs[d, pl.ds(lanes * c, lanes)] = plsc.load_gather(
                        tbl_v, [r, dv])
                    ews[d, pl.ds(lanes * c, lanes)] = plsc.load_gather(
                        tbl_v, [r + (2 * Hs - 1), dv])
            st_h = pltpu.async_copy(ehs, eh_hbm.at[wid], sem_h)
            st_w = pltpu.async_copy(ews, ew_hbm.at[wid], sem_w)
            st_h.wait()
            st_w.wait()

    return gather_kernel(tables)


def _expand_and_store(ehm, ewm, out_ref, Hs, Ws):
    HW = Hs * Ws
    colh = lax.broadcasted_iota(jnp.int32, (Hs, HW), 1) // Ws
    rowh = lax.broadcasted_iota(jnp.int32, (Hs, HW), 0)
    colw = lax.broadcasted_iota(jnp.int32, (Ws, HW), 1) % Ws
    roww = lax.broadcasted_iota(jnp.int32, (Ws, HW), 0)
    PT = (colh == rowh).astype(jnp.float32)
    QT = (colw == roww).astype(jnp.float32)
    EHb = jnp.dot(ehm, PT, preferred_element_type=jnp.float32)
    EWb = jnp.dot(ewm, QT, preferred_element_type=jnp.float32)
    for i in range(Hs):
        out_ref[0, pl.ds(i * Ws, Ws), :] = EHb[i:i + 1, :] + EWb


def _tc_self_expand(rhT3, rwT3, dim, dim_a, Hs, Ws):
    HW = Hs * Ws

    def body(rh_ref, rw_ref, out_ref):
        d = pl.program_id(0)
        ih = lax.broadcasted_iota(jnp.int32, (Hs, Hs), 0)
        kh = lax.broadcasted_iota(jnp.int32, (Hs, Hs), 1)
        idxh = ih - kh + (Hs - 1)
        iw = lax.broadcasted_iota(jnp.int32, (Ws, Ws), 0)
        lw = lax.broadcasted_iota(jnp.int32, (Ws, Ws), 1)
        idxw = iw - lw + (Ws - 1)
        ehm = jnp.zeros((Hs, Hs), jnp.float32)
        ewm = jnp.zeros((Ws, Ws), jnp.float32)
        for t in range(2 * Hs - 1):
            ehm = jnp.where(idxh == t, rh_ref[d, 0, t], ehm)
        for t in range(2 * Ws - 1):
            ewm = jnp.where(idxw == t, rw_ref[d, 0, t], ewm)
        _expand_and_store(ehm, ewm, out_ref, Hs, Ws)

    return pl.pallas_call(
        body,
        grid=(dim_a,),
        in_specs=[
            pl.BlockSpec(memory_space=pltpu.SMEM),
            pl.BlockSpec(memory_space=pltpu.SMEM),
        ],
        out_specs=pl.BlockSpec((1, HW, HW), lambda d: (d, 0, 0)),
        out_shape=jax.ShapeDtypeStruct((dim, HW, HW), jnp.float32),
    )(rhT3, rwT3)


def _tc_expand_rest(eh4, ew4, buf, dim, dim_a, Hs, Ws):
    HW = Hs * Ws

    def body(eh_ref, ew_ref, buf_ref, out_ref):
        del buf_ref
        ehm = eh_ref[:, 0, 0, :]
        ewm = ew_ref[:, 0, 0, :]
        _expand_and_store(ehm, ewm, out_ref, Hs, Ws)

    return pl.pallas_call(
        body,
        grid=(dim - dim_a,),
        in_specs=[
            pl.BlockSpec((Hs, 1, 1, Hs), lambda d: (0, d + dim_a, 0, 0)),
            pl.BlockSpec((Ws, 1, 1, Ws), lambda d: (0, d + dim_a, 0, 0)),
            pl.BlockSpec(memory_space=pl.ANY),
        ],
        out_specs=pl.BlockSpec((1, HW, HW), lambda d: (d + dim_a, 0, 0)),
        out_shape=jax.ShapeDtypeStruct((dim, HW, HW), jnp.float32),
        input_output_aliases={2: 0},
    )(eh4, ew4, buf)


def kernel(H, W, rel_height, rel_width):
    del H, W
    dim = rel_height.shape[1]
    Hs = (rel_height.shape[0] + 1) // 2
    Ws = (rel_width.shape[0] + 1) // 2
    dim_a = dim // 2
    tables = jnp.concatenate([rel_height, rel_width], axis=0)
    eh_sc, ew_sc = _sc_gather(tables, dim, Hs, Ws)
    eh4 = eh_sc.reshape(Hs, dim, 1, Hs)
    ew4 = ew_sc.reshape(Ws, dim, 1, Ws)
    rhT3 = jnp.transpose(rel_height)[:, None, :]
    rwT3 = jnp.transpose(rel_width)[:, None, :]
    buf = _tc_self_expand(rhT3, rwT3, dim, dim_a, Hs, Ws)
    out = _tc_expand_rest(eh4, ew4, buf, dim, dim_a, Hs, Ws)
    return out[None]


# speedup vs baseline: 1.0030x; 1.0030x over previous
import functools

import jax
import jax.numpy as jnp
from jax import lax
from jax.experimental import pallas as pl
from jax.experimental.pallas import tpu as pltpu
from jax.experimental.pallas import tpu_sc as plsc


def _sc_gather(tables, dim, Hs, Ws):
    lanes = 16
    nrows = tables.shape[0]
    mesh = plsc.VectorSubcoreMesh(
        core_axis_name="c", subcore_axis_name="s", num_cores=1)

    @functools.partial(
        pl.kernel,
        mesh=mesh,
        compiler_params=pltpu.CompilerParams(needs_layout_passes=False),
        out_type=(
            jax.ShapeDtypeStruct((Hs, dim, Hs), jnp.float32),
            jax.ShapeDtypeStruct((Ws, dim, Ws), jnp.float32),
        ),
        scratch_types=[
            pltpu.VMEM((nrows, dim), jnp.float32),
            pltpu.VMEM((dim, Hs), jnp.float32),
            pltpu.VMEM((dim, Ws), jnp.float32),
            pltpu.SemaphoreType.DMA,
            pltpu.SemaphoreType.DMA,
        ],
    )
    def gather_kernel(tbl_hbm, eh_hbm, ew_hbm, tbl_v, ehs, ews, sem_h, sem_w):
        sid = lax.axis_index("s")
        pltpu.async_copy(tbl_hbm, tbl_v, sem_h).wait()
        lane = lax.iota(jnp.int32, lanes)
        for half in range(2):
            wid = sid + 16 * half
            for c in range(Hs // lanes):
                r = (Hs - 1 + wid - lanes * c) - lane
                for d in range(dim):
                    dv = jnp.full((lanes,), d, jnp.int32)
                    ehs[d, pl.ds(lanes * c, lanes)] = plsc.load_gather(
                        tbl_v, [r, dv])
                    ews[d, pl.ds(lanes * c, lanes)] = plsc.load_gather(
                        tbl_v, [r + (2 * Hs - 1), dv])
            st_h = pltpu.async_copy(ehs, eh_hbm.at[wid], sem_h)
            st_w = pltpu.async_copy(ews, ew_hbm.at[wid], sem_w)
            st_h.wait()
            st_w.wait()

    return gather_kernel(tables)


def _expand_and_store(ehm, ewm, out_ref, Hs, Ws):
    HW = Hs * Ws
    colh = lax.broadcasted_iota(jnp.int32, (Hs, HW), 1) // Ws
    rowh = lax.broadcasted_iota(jnp.int32, (Hs, HW), 0)
    colw = lax.broadcasted_iota(jnp.int32, (Ws, HW), 1) % Ws
    roww = lax.broadcasted_iota(jnp.int32, (Ws, HW), 0)
    PT = (colh == rowh).astype(jnp.float32)
    QT = (colw == roww).astype(jnp.float32)
    EHb = jnp.dot(ehm, PT, preferred_element_type=jnp.float32)
    EWb = jnp.dot(ewm, QT, preferred_element_type=jnp.float32)
    for i in range(Hs):
        out_ref[0, pl.ds(i * Ws, Ws), :] = EHb[i:i + 1, :] + EWb


def _tc_self_expand(rhT3, rwT3, dim, dim_a, Hs, Ws):
    HW = Hs * Ws

    def body(rh_ref, rw_ref, out_ref):
        d = pl.program_id(0)
        ih = lax.broadcasted_iota(jnp.int32, (Hs, Hs), 0)
        kh = lax.broadcasted_iota(jnp.int32, (Hs, Hs), 1)
        idxh = ih - kh + (Hs - 1)
        iw = lax.broadcasted_iota(jnp.int32, (Ws, Ws), 0)
        lw = lax.broadcasted_iota(jnp.int32, (Ws, Ws), 1)
        idxw = iw - lw + (Ws - 1)
        ehm = jnp.zeros((Hs, Hs), jnp.float32)
        ewm = jnp.zeros((Ws, Ws), jnp.float32)
        for t in range(2 * Hs - 1):
            ehm = jnp.where(idxh == t, rh_ref[d, 0, t], ehm)
        for t in range(2 * Ws - 1):
            ewm = jnp.where(idxw == t, rw_ref[d, 0, t], ewm)
        _expand_and_store(ehm, ewm, out_ref, Hs, Ws)

    return pl.pallas_call(
        body,
        grid=(dim_a,),
        in_specs=[
            pl.BlockSpec(memory_space=pltpu.SMEM),
            pl.BlockSpec(memory_space=pltpu.SMEM),
        ],
        out_specs=pl.BlockSpec((1, HW, HW), lambda d: (d, 0, 0)),
        out_shape=jax.ShapeDtypeStruct((dim, HW, HW), jnp.float32),
    )(rhT3, rwT3)


def _tc_expand_rest(eh4, ew4, buf, dim, dim_a, Hs, Ws):
    HW = Hs * Ws

    def body(eh_ref, ew_ref, buf_ref, out_ref):
        del buf_ref
        ehm = eh_ref[:, 0, 0, :]
        ewm = ew_ref[:, 0, 0, :]
        _expand_and_store(ehm, ewm, out_ref, Hs, Ws)

    return pl.pallas_call(
        body,
        grid=(dim - dim_a,),
        in_specs=[
            pl.BlockSpec((Hs, 1, 1, Hs), lambda d: (0, d + dim_a, 0, 0)),
            pl.BlockSpec((Ws, 1, 1, Ws), lambda d: (0, d + dim_a, 0, 0)),
            pl.BlockSpec(memory_space=pl.ANY),
        ],
        out_specs=pl.BlockSpec((1, HW, HW), lambda d: (d + dim_a, 0, 0)),
        out_shape=jax.ShapeDtypeStruct((dim, HW, HW), jnp.float32),
        input_output_aliases={2: 0},
    )(eh4, ew4, buf)


def kernel(H, W, rel_height, rel_width):
    del H, W
    dim = rel_height.shape[1]
    Hs = (rel_height.shape[0] + 1) // 2
    Ws = (rel_width.shape[0] + 1) // 2
    dim_a = (10 * dim) // 16
    tables = jnp.concatenate([rel_height, rel_width], axis=0)
    eh_sc, ew_sc = _sc_gather(tables, dim, Hs, Ws)
    eh4 = eh_sc.reshape(Hs, dim, 1, Hs)
    ew4 = ew_sc.reshape(Ws, dim, 1, Ws)
    rhT3 = jnp.transpose(rel_height)[:, None, :]
    rwT3 = jnp.transpose(rel_width)[:, None, :]
    buf = _tc_self_expand(rhT3, rwT3, dim, dim_a, Hs, Ws)
    out = _tc_expand_rest(eh4, ew4, buf, dim, dim_a, Hs, Ws)
    return out[None]


# speedup vs baseline: 1.0632x; 1.0600x over previous
import functools

import jax
import jax.numpy as jnp
from jax import lax
from jax.experimental import pallas as pl
from jax.experimental.pallas import tpu as pltpu
from jax.experimental.pallas import tpu_sc as plsc


def _sc_gather(tables, dim, Hs, Ws):
    lanes = 16
    nrows = tables.shape[0]
    ncols = tables.shape[1]
    mesh = plsc.VectorSubcoreMesh(
        core_axis_name="c", subcore_axis_name="s", num_cores=1)

    @functools.partial(
        pl.kernel,
        mesh=mesh,
        compiler_params=pltpu.CompilerParams(needs_layout_passes=False),
        out_type=(
            jax.ShapeDtypeStruct((Hs, dim, Hs), jnp.float32),
            jax.ShapeDtypeStruct((Ws, dim, Ws), jnp.float32),
        ),
        scratch_types=[
            pltpu.VMEM((nrows, ncols), jnp.float32),
            pltpu.VMEM((dim, Hs), jnp.float32),
            pltpu.VMEM((dim, Ws), jnp.float32),
            pltpu.SemaphoreType.DMA,
            pltpu.SemaphoreType.DMA,
        ],
    )
    def gather_kernel(tbl_hbm, eh_hbm, ew_hbm, tbl_v, ehs, ews, sem_h, sem_w):
        sid = lax.axis_index("s")
        pltpu.async_copy(tbl_hbm, tbl_v, sem_h).wait()
        lane = lax.iota(jnp.int32, lanes)
        for half in range(2):
            wid = sid + 16 * half
            for c in range(Hs // lanes):
                r = (Hs - 1 + wid - lanes * c) - lane
                for d in range(dim):
                    dv = jnp.full((lanes,), d, jnp.int32)
                    ehs[d, pl.ds(lanes * c, lanes)] = plsc.load_gather(
                        tbl_v, [r, dv])
                    ews[d, pl.ds(lanes * c, lanes)] = plsc.load_gather(
                        tbl_v, [r, dv + dim])
            st_h = pltpu.async_copy(ehs, eh_hbm.at[wid], sem_h)
            st_w = pltpu.async_copy(ews, ew_hbm.at[wid], sem_w)
            st_h.wait()
            st_w.wait()

    return gather_kernel(tables)


def _expand_and_store(ehm, ewm, out_ref, Hs, Ws):
    HW = Hs * Ws
    colh = lax.broadcasted_iota(jnp.int32, (Hs, HW), 1) // Ws
    rowh = lax.broadcasted_iota(jnp.int32, (Hs, HW), 0)
    colw = lax.broadcasted_iota(jnp.int32, (Ws, HW), 1) % Ws
    roww = lax.broadcasted_iota(jnp.int32, (Ws, HW), 0)
    PT = (colh == rowh).astype(jnp.float32)
    QT = (colw == roww).astype(jnp.float32)
    EHb = jnp.dot(ehm, PT, preferred_element_type=jnp.float32)
    EWb = jnp.dot(ewm, QT, preferred_element_type=jnp.float32)
    for i in range(Hs):
        out_ref[0, pl.ds(i * Ws, Ws), :] = EHb[i:i + 1, :] + EWb


def _tc_self_expand(rhT3, rwT3, dim, dim_a, Hs, Ws):
    HW = Hs * Ws

    def body(rh_ref, rw_ref, out_ref):
        d = pl.program_id(0)
        ih = lax.broadcasted_iota(jnp.int32, (Hs, Hs), 0)
        kh = lax.broadcasted_iota(jnp.int32, (Hs, Hs), 1)
        idxh = ih - kh + (Hs - 1)
        iw = lax.broadcasted_iota(jnp.int32, (Ws, Ws), 0)
        lw = lax.broadcasted_iota(jnp.int32, (Ws, Ws), 1)
        idxw = iw - lw + (Ws - 1)
        ehm = jnp.zeros((Hs, Hs), jnp.float32)
        ewm = jnp.zeros((Ws, Ws), jnp.float32)
        for t in range(2 * Hs - 1):
            ehm = jnp.where(idxh == t, rh_ref[d, 0, t], ehm)
        for t in range(2 * Ws - 1):
            ewm = jnp.where(idxw == t, rw_ref[d, 0, t], ewm)
        _expand_and_store(ehm, ewm, out_ref, Hs, Ws)

    return pl.pallas_call(
        body,
        grid=(dim_a,),
        in_specs=[
            pl.BlockSpec(memory_space=pltpu.SMEM),
            pl.BlockSpec(memory_space=pltpu.SMEM),
        ],
        out_specs=pl.BlockSpec((1, HW, HW), lambda d: (d, 0, 0)),
        out_shape=jax.ShapeDtypeStruct((dim, HW, HW), jnp.float32),
    )(rhT3, rwT3)


def _tc_expand_rest(eh4, ew4, buf, dim, dim_a, Hs, Ws):
    HW = Hs * Ws

    def body(eh_ref, ew_ref, buf_ref, out_ref):
        del buf_ref
        ehm = eh_ref[:, 0, 0, :]
        ewm = ew_ref[:, 0, 0, :]
        _expand_and_store(ehm, ewm, out_ref, Hs, Ws)

    return pl.pallas_call(
        body,
        grid=(dim - dim_a,),
        in_specs=[
            pl.BlockSpec((Hs, 1, 1, Hs), lambda d: (0, d + dim_a, 0, 0)),
            pl.BlockSpec((Ws, 1, 1, Ws), lambda d: (0, d + dim_a, 0, 0)),
            pl.BlockSpec(memory_space=pl.ANY),
        ],
        out_specs=pl.BlockSpec((1, HW, HW), lambda d: (d + dim_a, 0, 0)),
        out_shape=jax.ShapeDtypeStruct((dim, HW, HW), jnp.float32),
        input_output_aliases={2: 0},
    )(eh4, ew4, buf)


def kernel(H, W, rel_height, rel_width):
    del H, W
    dim = rel_height.shape[1]
    Hs = (rel_height.shape[0] + 1) // 2
    Ws = (rel_width.shape[0] + 1) // 2
    dim_a = (10 * dim) // 16
    tables = jnp.concatenate([rel_height, rel_width], axis=1)
    eh_sc, ew_sc = _sc_gather(tables, dim, Hs, Ws)
    eh4 = eh_sc.reshape(Hs, dim, 1, Hs)
    ew4 = ew_sc.reshape(Ws, dim, 1, Ws)
    rhT3 = jnp.transpose(rel_height)[:, None, :]
    rwT3 = jnp.transpose(rel_width)[:, None, :]
    buf = _tc_self_expand(rhT3, rwT3, dim, dim_a, Hs, Ws)
    out = _tc_expand_rest(eh4, ew4, buf, dim, dim_a, Hs, Ws)
    return out[None]


# speedup vs baseline: 1.0669x; 1.0035x over previous
import functools

import jax
import jax.numpy as jnp
from jax import lax
from jax.experimental import pallas as pl
from jax.experimental.pallas import tpu as pltpu
from jax.experimental.pallas import tpu_sc as plsc


def _sc_gather(tables, dim, Hs, Ws):
    lanes = 16
    nrows = tables.shape[0]
    ncols = tables.shape[1]
    mesh = plsc.VectorSubcoreMesh(
        core_axis_name="c", subcore_axis_name="s", num_cores=1)

    @functools.partial(
        pl.kernel,
        mesh=mesh,
        compiler_params=pltpu.CompilerParams(needs_layout_passes=False),
        out_type=(
            jax.ShapeDtypeStruct((Hs, dim, Hs), jnp.float32),
            jax.ShapeDtypeStruct((Ws, dim, Ws), jnp.float32),
        ),
        scratch_types=[
            pltpu.VMEM((nrows, ncols), jnp.float32),
            pltpu.VMEM((dim, Hs), jnp.float32),
            pltpu.VMEM((dim, Ws), jnp.float32),
            pltpu.SemaphoreType.DMA,
            pltpu.SemaphoreType.DMA,
        ],
    )
    def gather_kernel(tbl_hbm, eh_hbm, ew_hbm, tbl_v, ehs, ews, sem_h, sem_w):
        sid = lax.axis_index("s")
        pltpu.async_copy(tbl_hbm, tbl_v, sem_h).wait()
        lane = lax.iota(jnp.int32, lanes)
        for half in range(2):
            wid = sid + 16 * half
            for c in range(Hs // lanes):
                r = (Hs - 1 + wid - lanes * c) - lane
                for d in range(dim):
                    dv = jnp.full((lanes,), d, jnp.int32)
                    ehs[d, pl.ds(lanes * c, lanes)] = plsc.load_gather(
                        tbl_v, [r, dv])
                    ews[d, pl.ds(lanes * c, lanes)] = plsc.load_gather(
                        tbl_v, [r, dv + dim])
            st_h = pltpu.async_copy(ehs, eh_hbm.at[wid], sem_h)
            st_w = pltpu.async_copy(ews, ew_hbm.at[wid], sem_w)
            st_h.wait()
            st_w.wait()

    return gather_kernel(tables)


def _expand_and_store(ehm, ewm, out_ref, Hs, Ws):
    HW = Hs * Ws
    colh = lax.broadcasted_iota(jnp.int32, (Hs, HW), 1) // Ws
    rowh = lax.broadcasted_iota(jnp.int32, (Hs, HW), 0)
    colw = lax.broadcasted_iota(jnp.int32, (Ws, HW), 1) % Ws
    roww = lax.broadcasted_iota(jnp.int32, (Ws, HW), 0)
    PT = (colh == rowh).astype(jnp.float32)
    QT = (colw == roww).astype(jnp.float32)
    EHb = jnp.dot(ehm, PT, preferred_element_type=jnp.float32)
    EWb = jnp.dot(ewm, QT, preferred_element_type=jnp.float32)
    for i in range(Hs):
        out_ref[0, pl.ds(i * Ws, Ws), :] = EHb[i:i + 1, :] + EWb


def _tc_self_expand(rhT3, rwT3, dim, dim_a, Hs, Ws):
    HW = Hs * Ws

    def body(rh_ref, rw_ref, out_ref):
        ih = lax.broadcasted_iota(jnp.int32, (Hs, Hs), 0)
        kh = lax.broadcasted_iota(jnp.int32, (Hs, Hs), 1)
        idxh = ih - kh + (Hs - 1)
        iw = lax.broadcasted_iota(jnp.int32, (Ws, Ws), 0)
        lw = lax.broadcasted_iota(jnp.int32, (Ws, Ws), 1)
        idxw = iw - lw + (Ws - 1)
        for dd in range(2):
            d = pl.program_id(0) * 2 + dd
            ehm = jnp.zeros((Hs, Hs), jnp.float32)
            ewm = jnp.zeros((Ws, Ws), jnp.float32)
            for t in range(2 * Hs - 1):
                ehm = jnp.where(idxh == t, rh_ref[d, 0, t], ehm)
            for t in range(2 * Ws - 1):
                ewm = jnp.where(idxw == t, rw_ref[d, 0, t], ewm)
            _expand_and_store(ehm, ewm, out_ref.at[pl.ds(dd, 1)], Hs, Ws)

    return pl.pallas_call(
        body,
        grid=(dim_a // 2,),
        in_specs=[
            pl.BlockSpec(memory_space=pltpu.SMEM),
            pl.BlockSpec(memory_space=pltpu.SMEM),
        ],
        out_specs=pl.BlockSpec((2, HW, HW), lambda d: (d, 0, 0)),
        out_shape=jax.ShapeDtypeStruct((dim, HW, HW), jnp.float32),
    )(rhT3, rwT3)


def _tc_expand_rest(eh4, ew4, buf, dim, dim_a, Hs, Ws):
    HW = Hs * Ws

    def body(eh_ref, ew_ref, buf_ref, out_ref):
        del buf_ref
        for dd in range(2):
            ehm = eh_ref[:, dd, 0, :]
            ewm = ew_ref[:, dd, 0, :]
            _expand_and_store(ehm, ewm, out_ref.at[pl.ds(dd, 1)], Hs, Ws)

    return pl.pallas_call(
        body,
        grid=((dim - dim_a) // 2,),
        in_specs=[
            pl.BlockSpec((Hs, 2, 1, Hs), lambda d: (0, d + dim_a // 2, 0, 0)),
            pl.BlockSpec((Ws, 2, 1, Ws), lambda d: (0, d + dim_a // 2, 0, 0)),
            pl.BlockSpec(memory_space=pl.ANY),
        ],
        out_specs=pl.BlockSpec((2, HW, HW), lambda d: (d + dim_a // 2, 0, 0)),
        out_shape=jax.ShapeDtypeStruct((dim, HW, HW), jnp.float32),
        input_output_aliases={2: 0},
    )(eh4, ew4, buf)


def kernel(H, W, rel_height, rel_width):
    del H, W
    dim = rel_height.shape[1]
    Hs = (rel_height.shape[0] + 1) // 2
    Ws = (rel_width.shape[0] + 1) // 2
    dim_a = (10 * dim) // 16
    tables = jnp.concatenate([rel_height, rel_width], axis=1)
    eh_sc, ew_sc = _sc_gather(tables, dim, Hs, Ws)
    eh4 = eh_sc.reshape(Hs, dim, 1, Hs)
    ew4 = ew_sc.reshape(Ws, dim, 1, Ws)
    rhT3 = jnp.transpose(rel_height)[:, None, :]
    rwT3 = jnp.transpose(rel_width)[:, None, :]
    buf = _tc_self_expand(rhT3, rwT3, dim, dim_a, Hs, Ws)
    out = _tc_expand_rest(eh4, ew4, buf, dim, dim_a, Hs, Ws)
    return out[None]
